# initial kernel scaffold (unmeasured)
import jax
import jax.numpy as jnp
from jax import lax
from jax.experimental import pallas as pl
from jax.experimental.pallas import tpu as pltpu


def kernel(
    x,
):
    def body(*refs):
        pass

    out_shape = jax.ShapeDtypeStruct(..., jnp.float32)
    return pl.pallas_call(body, out_shape=out_shape)(...)


# baseline (device time: 424314 ns/iter reference)
import jax
import jax.numpy as jnp
from jax import lax
from jax.experimental import pallas as pl
from jax.experimental.pallas import tpu as pltpu

CHUNK = 1024


def kernel(x):
    _, m, n2 = x.shape
    n = n2 // 2
    n_chunks = m // CHUNK

    def body(x_ref, out_ref, recv_ref, a_ref, b_ref, o_ref,
             local_sems, send_sem, recv_sem):
        mx = lax.axis_index("x")
        my = lax.axis_index("y")
        mz = lax.axis_index("z")
        peer = (mx, my, 1 - mz)

        barrier_sem = pltpu.get_barrier_semaphore()
        pl.semaphore_signal(barrier_sem, inc=1, device_id=peer,
                            device_id_type=pl.DeviceIdType.MESH)
        pl.semaphore_wait(barrier_sem, 1)

        peer_col = (1 - mz) * n
        my_col = mz * n

        rdma = pltpu.make_async_remote_copy(
            src_ref=x_ref.at[0, :, pl.ds(peer_col, n)],
            dst_ref=recv_ref,
            send_sem=send_sem,
            recv_sem=recv_sem,
            device_id=peer,
            device_id_type=pl.DeviceIdType.MESH,
        )
        rdma.start()
        rdma.wait()

        for i in range(n_chunks):
            r0 = i * CHUNK
            ca = pltpu.make_async_copy(
                x_ref.at[0, pl.ds(r0, CHUNK), pl.ds(my_col, n)],
                a_ref, local_sems.at[0])
            cb = pltpu.make_async_copy(
                recv_ref.at[pl.ds(r0, CHUNK), :], b_ref, local_sems.at[1])
            ca.start()
            cb.start()
            ca.wait()
            cb.wait()
            o_ref[...] = a_ref[...] + b_ref[...]
            co = pltpu.make_async_copy(
                o_ref, out_ref.at[pl.ds(r0, CHUNK), :], local_sems.at[2])
            co.start()
            co.wait()

    return pl.pallas_call(
        body,
        out_shape=jax.ShapeDtypeStruct((m, n), jnp.float32),
        in_specs=[pl.BlockSpec(memory_space=pl.ANY)],
        out_specs=pl.BlockSpec(memory_space=pl.ANY),
        scratch_shapes=[
            pltpu.VMEM((m, n), jnp.float32),
            pltpu.VMEM((CHUNK, n), jnp.float32),
            pltpu.VMEM((CHUNK, n), jnp.float32),
            pltpu.VMEM((CHUNK, n), jnp.float32),
            pltpu.SemaphoreType.DMA((3,)),
            pltpu.SemaphoreType.DMA,
            pltpu.SemaphoreType.DMA,
        ],
        compiler_params=pltpu.CompilerParams(
            collective_id=0, vmem_limit_bytes=100 * 1024 * 1024),
    )(x)


# device time: 339110 ns/iter; 1.2513x vs baseline; 1.2513x over previous
import jax
import jax.numpy as jnp
from jax import lax
from jax.experimental import pallas as pl
from jax.experimental.pallas import tpu as pltpu

CHUNK = 1024


def _fp_coords(q):
    q = q % 4
    qx = q // 2
    qy = jnp.logical_or(q == 1, q == 2).astype(q.dtype)
    return qx, qy


def kernel(x):
    _, m, n2 = x.shape
    n = n2 // 2
    g = m // 4
    n_chunks = m // CHUNK

    def body(x_ref, out_ref, r_ref, a_ref, o_ref, local_sems, dma_sems):
        mx = lax.axis_index("x")
        my = lax.axis_index("y")
        mz = lax.axis_index("z")
        fp = mx * 2 + jnp.bitwise_xor(my, mx)

        zpeer = (mx, my, 1 - mz)
        rx, ry = _fp_coords(fp + 1)
        lx, ly = _fp_coords(fp - 1)
        right = (rx, ry, mz)
        left = (lx, ly, mz)

        Z_S, Z_R, AR_S, AR_R, AL_S, AL_R, B_S, B_R = range(8)

        barrier_sem = pltpu.get_barrier_semaphore()
        for nbr in (zpeer, left, right):
            pl.semaphore_signal(barrier_sem, inc=1, device_id=nbr,
                                device_id_type=pl.DeviceIdType.MESH)
        pl.semaphore_wait(barrier_sem, 3)

        peer_col = (1 - mz) * n

        zc = pltpu.make_async_remote_copy(
            src_ref=x_ref.at[0, pl.ds(fp * g, g), pl.ds(peer_col, n)],
            dst_ref=r_ref.at[pl.ds(fp * g, g), :],
            send_sem=dma_sems.at[Z_S], recv_sem=dma_sems.at[Z_R],
            device_id=zpeer, device_id_type=pl.DeviceIdType.MESH,
        )
        zc.start()
        zc.wait()

        ar = pltpu.make_async_remote_copy(
            src_ref=r_ref.at[pl.ds(fp * g, g), :],
            dst_ref=r_ref.at[pl.ds(fp * g, g), :],
            send_sem=dma_sems.at[AR_S], recv_sem=dma_sems.at[AR_R],
            device_id=right, device_id_type=pl.DeviceIdType.MESH,
        )
        al = pltpu.make_async_remote_copy(
            src_ref=r_ref.at[pl.ds(fp * g, g), :],
            dst_ref=r_ref.at[pl.ds(fp * g, g), :],
            send_sem=dma_sems.at[AL_S], recv_sem=dma_sems.at[AL_R],
            device_id=left, device_id_type=pl.DeviceIdType.MESH,
        )
        ar.start()
        al.start()
        ar.wait()
        al.wait()

        pm1 = (fp - 1) % 4
        bf = pltpu.make_async_remote_copy(
            src_ref=r_ref.at[pl.ds(pm1 * g, g), :],
            dst_ref=r_ref.at[pl.ds(pm1 * g, g), :],
            send_sem=dma_sems.at[B_S], recv_sem=dma_sems.at[B_R],
            device_id=right, device_id_type=pl.DeviceIdType.MESH,
        )
        bf.start()
        bf.wait()

        my_col = mz * n
        for i in range(n_chunks):
            r0 = i * CHUNK
            ca = pltpu.make_async_copy(
                x_ref.at[0, pl.ds(r0, CHUNK), pl.ds(my_col, n)],
                a_ref, local_sems.at[0])
            ca.start()
            ca.wait()
            o_ref[...] = a_ref[...] + r_ref[r0:r0 + CHUNK, :]
            co = pltpu.make_async_copy(
                o_ref, out_ref.at[pl.ds(r0, CHUNK), :], local_sems.at[1])
            co.start()
            co.wait()

    return pl.pallas_call(
        body,
        out_shape=jax.ShapeDtypeStruct((m, n), jnp.float32),
        in_specs=[pl.BlockSpec(memory_space=pl.ANY)],
        out_specs=pl.BlockSpec(memory_space=pl.ANY),
        scratch_shapes=[
            pltpu.VMEM((m, n), jnp.float32),
            pltpu.VMEM((CHUNK, n), jnp.float32),
            pltpu.VMEM((CHUNK, n), jnp.float32),
            pltpu.SemaphoreType.DMA((2,)),
            pltpu.SemaphoreType.DMA((8,)),
        ],
        compiler_params=pltpu.CompilerParams(
            collective_id=0, vmem_limit_bytes=100 * 1024 * 1024),
    )(x)


# device time: 186876 ns/iter; 2.2706x vs baseline; 1.8146x over previous
import jax
import jax.numpy as jnp
from jax import lax
from jax.experimental import pallas as pl
from jax.experimental.pallas import tpu as pltpu

CHUNK = 1024
S = 8


def _fp_coords(q):
    q = q % 4
    qx = q // 2
    qy = jnp.logical_or(q == 1, q == 2).astype(q.dtype)
    return qx, qy


def kernel(x):
    _, m, n2 = x.shape
    n = n2 // 2
    g = m // 4
    sub = g // S

    def body(x_ref, out_ref, r_ref, a_ref, o_ref, a_sems, o_sems,
             z_sems, ar_sems, al_sems, b_sems):
        mx = lax.axis_index("x")
        my = lax.axis_index("y")
        mz = lax.axis_index("z")
        fp = mx * 2 + jnp.bitwise_xor(my, mx)

        zpeer = (mx, my, 1 - mz)
        rx, ry = _fp_coords(fp + 1)
        lx, ly = _fp_coords(fp - 1)
        right = (rx, ry, mz)
        left = (lx, ly, mz)
        pm1 = (fp - 1) % 4
        pp1 = (fp + 1) % 4
        pp2 = (fp + 2) % 4

        barrier_sem = pltpu.get_barrier_semaphore()
        for nbr in (zpeer, left, right):
            pl.semaphore_signal(barrier_sem, inc=1, device_id=nbr,
                                device_id_type=pl.DeviceIdType.MESH)
        pl.semaphore_wait(barrier_sem, 3)

        peer_col = (1 - mz) * n
        my_col = mz * n

        def rcopy(row0, dev, send_sem, recv_sem, src_x=False):
            sl = pl.ds(row0, sub)
            src = (x_ref.at[0, sl, pl.ds(peer_col, n)] if src_x
                   else r_ref.at[sl, :])
            return pltpu.make_async_remote_copy(
                src_ref=src, dst_ref=r_ref.at[sl, :],
                send_sem=send_sem, recv_sem=recv_sem,
                device_id=dev, device_id_type=pl.DeviceIdType.MESH,
            )

        zc = []
        for s in range(S):
            c = rcopy(fp * g + s * sub, zpeer,
                      z_sems.at[0, s], z_sems.at[1, s], src_x=True)
            c.start()
            zc.append(c)

        arc, alc = [], []
        for s in range(S):
            zc[s].wait_recv()
            c = rcopy(fp * g + s * sub, right,
                      ar_sems.at[0, s], ar_sems.at[1, s])
            c.start()
            arc.append(c)
            c = rcopy(fp * g + s * sub, left,
                      al_sems.at[0, s], al_sems.at[1, s])
            c.start()
            alc.append(c)

        pending_o = [None, None]
        state = {"idx": 0}

        def add_rows(row0):
            slot = state["idx"] % 2
            state["idx"] += 1
            ca = pltpu.make_async_copy(
                x_ref.at[0, pl.ds(row0, CHUNK), pl.ds(my_col, n)],
                a_ref.at[slot], a_sems.at[slot])
            ca.start()
            ca.wait()
            if pending_o[slot] is not None:
                pending_o[slot].wait()
            o_ref[slot] = a_ref[slot] + r_ref[pl.ds(row0, CHUNK), :]
            co = pltpu.make_async_copy(
                o_ref.at[slot], out_ref.at[pl.ds(row0, CHUNK), :],
                o_sems.at[slot])
            co.start()
            pending_o[slot] = co

        def add_group(q):
            for j in range(g // CHUNK):
                add_rows(q * g + j * CHUNK)

        add_group(fp)

        bc = []
        for s in range(S):
            arc[s].wait_recv()
            alc[s].wait_recv()
            if s % 2 == 0:
                c = rcopy(pm1 * g + s * sub, right,
                          b_sems.at[0, s], b_sems.at[1, s])
            else:
                c = rcopy(pp1 * g + s * sub, left,
                          b_sems.at[0, s], b_sems.at[1, s])
            c.start()
            bc.append(c)

        add_group(pm1)
        add_group(pp1)

        for s in range(S):
            bc[s].wait_recv()
        add_group(pp2)

        for s in range(S):
            zc[s].wait_send()
            arc[s].wait_send()
            alc[s].wait_send()
            bc[s].wait_send()
        for co in pending_o:
            if co is not None:
                co.wait()

    return pl.pallas_call(
        body,
        out_shape=jax.ShapeDtypeStruct((m, n), jnp.float32),
        in_specs=[pl.BlockSpec(memory_space=pl.ANY)],
        out_specs=pl.BlockSpec(memory_space=pl.ANY),
        scratch_shapes=[
            pltpu.VMEM((m, n), jnp.float32),
            pltpu.VMEM((2, CHUNK, n), jnp.float32),
            pltpu.VMEM((2, CHUNK, n), jnp.float32),
            pltpu.SemaphoreType.DMA((2,)),
            pltpu.SemaphoreType.DMA((2,)),
            pltpu.SemaphoreType.DMA((2, S)),
            pltpu.SemaphoreType.DMA((2, S)),
            pltpu.SemaphoreType.DMA((2, S)),
            pltpu.SemaphoreType.DMA((2, S)),
        ],
        compiler_params=pltpu.CompilerParams(
            collective_id=0, vmem_limit_bytes=100 * 1024 * 1024),
    )(x)


# device time: 174069 ns/iter; 2.4376x vs baseline; 1.0736x over previous
import jax
import jax.numpy as jnp
from jax import lax
from jax.experimental import pallas as pl
from jax.experimental.pallas import tpu as pltpu

CHUNK = 1024
S = 8
ZD = [s for s in range(S) if s % 3 == 0]
FR = [s for s in range(S) if s % 3 == 1]
FL = [s for s in range(S) if s % 3 == 2]


def _fp_coords(q):
    q = q % 4
    qx = q // 2
    qy = jnp.logical_or(q == 1, q == 2).astype(q.dtype)
    return qx, qy


def kernel(x):
    _, m, n2 = x.shape
    n = n2 // 2
    g = m // 4
    sub = g // S
    cs = CHUNK // sub

    def body(x_ref, out_ref, r_ref, a_ref, o_ref, a_sems, o_sems,
             z_sems, ar_sems, al_sems, b_sems):
        mx = lax.axis_index("x")
        my = lax.axis_index("y")
        mz = lax.axis_index("z")
        fp = mx * 2 + jnp.bitwise_xor(my, mx)

        zpeer = (mx, my, 1 - mz)
        rx, ry = _fp_coords(fp + 1)
        lx, ly = _fp_coords(fp - 1)
        right = (rx, ry, mz)
        left = (lx, ly, mz)
        pm1 = (fp - 1) % 4
        pp1 = (fp + 1) % 4
        pp2 = (fp + 2) % 4

        barrier_sem = pltpu.get_barrier_semaphore()
        for nbr in (zpeer, left, right):
            pl.semaphore_signal(barrier_sem, inc=1, device_id=nbr,
                                device_id_type=pl.DeviceIdType.MESH)
        pl.semaphore_wait(barrier_sem, 3)

        peer_col = (1 - mz) * n
        my_col = mz * n

        def wait_in(row0, sem):
            sl = pl.ds(row0, sub)
            pltpu.make_async_remote_copy(
                src_ref=r_ref.at[sl, :], dst_ref=r_ref.at[sl, :],
                send_sem=sem, recv_sem=sem,
                device_id=zpeer, device_id_type=pl.DeviceIdType.MESH,
            ).wait_recv()

        def rcopy(row0, dev, send_sem, recv_sem, src_x=False):
            sl = pl.ds(row0, sub)
            src = (x_ref.at[0, sl, pl.ds(peer_col, n)] if src_x
                   else r_ref.at[sl, :])
            return pltpu.make_async_remote_copy(
                src_ref=src, dst_ref=r_ref.at[sl, :],
                send_sem=send_sem, recv_sem=recv_sem,
                device_id=dev, device_id_type=pl.DeviceIdType.MESH,
            )

        zc = []
        for s in range(S):
            c = rcopy(fp * g + s * sub, zpeer,
                      z_sems.at[0, s], z_sems.at[1, s], src_x=True)
            c.start()
            zc.append(c)
        zc2 = {}
        for k, s in enumerate(ZD):
            c = rcopy(pp2 * g + s * sub, zpeer,
                      z_sems.at[0, S + k], z_sems.at[1, S + k], src_x=True)
            c.start()
            zc2[s] = c

        pending_o = [None, None]
        aq = []
        state = {"ai": 0}

        def flush_one():
            ca, slot, row0 = aq.pop(0)
            ca.wait()
            if pending_o[slot] is not None:
                pending_o[slot].wait()
            o_ref[slot] = a_ref[slot] + r_ref[pl.ds(row0, CHUNK), :]
            co = pltpu.make_async_copy(
                o_ref.at[slot], out_ref.at[pl.ds(row0, CHUNK), :],
                o_sems.at[slot])
            co.start()
            pending_o[slot] = co

        def add_rows(row0):
            while len(aq) >= 2:
                flush_one()
            slot = state["ai"] % 2
            state["ai"] += 1
            ca = pltpu.make_async_copy(
                x_ref.at[0, pl.ds(row0, CHUNK), pl.ds(my_col, n)],
                a_ref.at[slot], a_sems.at[slot])
            ca.start()
            aq.append((ca, slot, row0))

        arc, alc = [], []
        for s in range(S):
            zc[s].wait_recv()
            c = rcopy(fp * g + s * sub, right,
                      ar_sems.at[0, s], ar_sems.at[1, s])
            c.start()
            arc.append(c)
            c = rcopy(fp * g + s * sub, left,
                      al_sems.at[0, s], al_sems.at[1, s])
            c.start()
            alc.append(c)
            if (s + 1) % cs == 0:
                add_rows(fp * g + (s + 1 - cs) * sub)

        bc = {}
        for s in range(S):
            wait_in(pm1 * g + s * sub, ar_sems.at[1, s])
            wait_in(pp1 * g + s * sub, al_sems.at[1, s])
            if s in FR:
                c = rcopy(pm1 * g + s * sub, right,
                          b_sems.at[0, s], b_sems.at[1, s])
                c.start()
                bc[s] = c
            elif s in FL:
                c = rcopy(pp1 * g + s * sub, left,
                          b_sems.at[0, s], b_sems.at[1, s])
                c.start()
                bc[s] = c
            if (s + 1) % cs == 0:
                add_rows(pm1 * g + (s + 1 - cs) * sub)
                add_rows(pp1 * g + (s + 1 - cs) * sub)

        for j in range(g // CHUNK):
            for s in range(j * cs, (j + 1) * cs):
                if s in ZD:
                    zc2[s].wait_recv()
                else:
                    wait_in(pp2 * g + s * sub, b_sems.at[1, s])
            add_rows(pp2 * g + j * CHUNK)

        while aq:
            flush_one()

        for s in range(S):
            zc[s].wait_send()
            arc[s].wait_send()
            alc[s].wait_send()
        for c in zc2.values():
            c.wait_send()
        for c in bc.values():
            c.wait_send()
        for co in pending_o:
            if co is not None:
                co.wait()

    return pl.pallas_call(
        body,
        out_shape=jax.ShapeDtypeStruct((m, n), jnp.float32),
        in_specs=[pl.BlockSpec(memory_space=pl.ANY)],
        out_specs=pl.BlockSpec(memory_space=pl.ANY),
        scratch_shapes=[
            pltpu.VMEM((m, n), jnp.float32),
            pltpu.VMEM((2, CHUNK, n), jnp.float32),
            pltpu.VMEM((2, CHUNK, n), jnp.float32),
            pltpu.SemaphoreType.DMA((2,)),
            pltpu.SemaphoreType.DMA((2,)),
            pltpu.SemaphoreType.DMA((2, S + len(ZD))),
            pltpu.SemaphoreType.DMA((2, S)),
            pltpu.SemaphoreType.DMA((2, S)),
            pltpu.SemaphoreType.DMA((2, S)),
        ],
        compiler_params=pltpu.CompilerParams(
            collective_id=0, vmem_limit_bytes=100 * 1024 * 1024),
    )(x)
